# Initial kernel scaffold; baseline (speedup 1.0000x reference)
import jax
import jax.numpy as jnp
from jax.experimental import pallas as pl


def kernel(x_user, x_item, ei_u2i, ei_i2u, W_l_l0_u2i, W_r_l0_u2i, b_l0_u2i, W_l_l0_i2u, W_r_l0_i2u, b_l0_i2u, W_l_l1_u2i, W_r_l1_u2i, b_l1_u2i, W_l_l1_i2u, W_r_l1_i2u, b_l1_i2u):
    raise NotImplementedError("write your pallas kernel here")


# speedup vs baseline: 3.4750x; 3.4750x over previous
import functools

import jax
import jax.numpy as jnp
from jax import lax
from jax.experimental import pallas as pl
from jax.experimental.pallas import tpu as pltpu
from jax.experimental.pallas import tpu_sc as plsc

N = 10000
D = 128
E = 320000
NTILE = 16
CHUNK = 128
K = 160
KB = 16
NSTAGE = K // KB
EPAD = NTILE * K * CHUNK
ROWS_PT = 632
NPAD = NTILE * ROWS_PT

_SPANS = []
_r = 0
while _r < ROWS_PT:
    _SPANS.append((_r, min(CHUNK, ROWS_PT - _r)))
    _r += CHUNK


def _sc_body(with_counts, *refs):
    if with_counts:
        (tab0, tab1, s0, d0, s1, d1, zfeat, ones_hbm,
         out0, out1, cnt0, cnt1,
         acc, sidx, didx, rows, sem) = refs
    else:
        (tab0, tab1, s0, d0, s1, d1, zfeat,
         out0, out1,
         acc, sidx, didx, rows, sem) = refs
        cnt0 = cnt1 = None

    core = lax.axis_index("c")
    sid = lax.axis_index("s")
    rbase = sid * ROWS_PT

    def zero_acc_slice():
        pltpu.sync_copy(zfeat, rows)
        for (o, l) in _SPANS:
            pltpu.sync_copy(rows.at[pl.ds(0, l)], acc.at[pl.ds(rbase + o, l)])

    def copy_out(dst_hbm):
        for (o, l) in _SPANS:
            pltpu.sync_copy(acc.at[pl.ds(rbase + o, l)], rows.at[pl.ds(0, l)])
            pltpu.sync_copy(rows.at[pl.ds(0, l)], dst_hbm.at[pl.ds(rbase + o, l)])

    def run_dir(tab, srcr, dstr, out, cntout):
        if with_counts:
            zero_acc_slice()
            pltpu.sync_copy(ones_hbm, rows)
            plsc.subcore_barrier()

            def cstage(s, carry):
                pltpu.sync_copy(dstr.at[pl.ds(sid * K + s * KB, KB)], didx)

                def cbody(j, c2):
                    pltpu.sync_copy(rows, acc.at[didx.at[j]], add=True)
                    return c2

                return lax.fori_loop(0, KB, cbody, carry)

            lax.fori_loop(0, NSTAGE, cstage, 0)
            plsc.subcore_barrier()
            copy_out(cntout)

        zero_acc_slice()
        plsc.subcore_barrier()

        def stage(s, carry):
            pltpu.sync_copy(srcr.at[pl.ds(sid * K + s * KB, KB)], sidx)
            pltpu.sync_copy(dstr.at[pl.ds(sid * K + s * KB, KB)], didx)

            def body(j, c2):
                pltpu.async_copy(tab.at[sidx.at[j]], rows, sem).wait()
                pltpu.sync_copy(rows, acc.at[didx.at[j]], add=True)
                return c2

            return lax.fori_loop(0, KB, body, carry)

        lax.fori_loop(0, NSTAGE, stage, 0)
        plsc.subcore_barrier()
        copy_out(out)

    @pl.when(core == 0)
    def _():
        run_dir(tab0, s0, d0, out0, cnt0)

    @pl.when(core == 1)
    def _():
        run_dir(tab1, s1, d1, out1, cnt1)


def _make_sc_call(with_counts):
    n_out = 4 if with_counts else 2
    out_type = [jax.ShapeDtypeStruct((NPAD, D), jnp.float32)] * n_out
    scratch = [
        pltpu.VMEM_SHARED((NPAD, D), jnp.float32),
        pltpu.VMEM((KB, CHUNK), jnp.int32),
        pltpu.VMEM((KB, CHUNK), jnp.int32),
        pltpu.VMEM((CHUNK, D), jnp.float32),
        pltpu.SemaphoreType.DMA,
    ]
    mesh = plsc.VectorSubcoreMesh(core_axis_name="c", subcore_axis_name="s")
    return pl.kernel(
        functools.partial(_sc_body, with_counts),
        out_type=out_type,
        mesh=mesh,
        scratch_types=scratch,
    )


_sc_layer0 = _make_sc_call(True)
_sc_layer1 = _make_sc_call(False)


def _tc_body(relu, agg_ref, cnt_ref, x_ref, wl_ref, wr_ref, b_ref, o_ref):
    c = jnp.maximum(cnt_ref[:, 0:1], 1.0)
    mean = agg_ref[...] / c
    acc = jnp.dot(mean, wl_ref[...], preferred_element_type=jnp.float32)
    acc = acc + jnp.dot(x_ref[...], wr_ref[...], preferred_element_type=jnp.float32)
    acc = acc + b_ref[...]
    if relu:
        acc = jnp.maximum(acc, 0.0)
    o_ref[...] = acc


_TC_BLK = 1000


def _tc_update(agg, cnt, x_dst, W_l, W_r, b, relu):
    grid = (N // _TC_BLK,)
    return pl.pallas_call(
        functools.partial(_tc_body, relu),
        grid=grid,
        in_specs=[
            pl.BlockSpec((_TC_BLK, D), lambda i: (i, 0)),
            pl.BlockSpec((_TC_BLK, D), lambda i: (i, 0)),
            pl.BlockSpec((_TC_BLK, D), lambda i: (i, 0)),
            pl.BlockSpec((D, D), lambda i: (0, 0)),
            pl.BlockSpec((D, D), lambda i: (0, 0)),
            pl.BlockSpec((1, D), lambda i: (0, 0)),
        ],
        out_specs=pl.BlockSpec((_TC_BLK, D), lambda i: (i, 0)),
        out_shape=jax.ShapeDtypeStruct((N, D), jnp.float32),
    )(agg, cnt, x_dst, W_l, W_r, b.reshape(1, D))


def _prep_edges(ei):
    src = ei[0].astype(jnp.int32)
    dst = ei[1].astype(jnp.int32)
    pad = EPAD - E
    src = jnp.concatenate([src, jnp.zeros((pad,), jnp.int32)])
    dst = jnp.concatenate([dst, jnp.full((pad,), N, jnp.int32)])
    return src.reshape(NTILE * K, CHUNK), dst.reshape(NTILE * K, CHUNK)


def kernel(x_user, x_item, ei_u2i, ei_i2u,
           W_l_l0_u2i, W_r_l0_u2i, b_l0_u2i,
           W_l_l0_i2u, W_r_l0_i2u, b_l0_i2u,
           W_l_l1_u2i, W_r_l1_u2i, b_l1_u2i,
           W_l_l1_i2u, W_r_l1_i2u, b_l1_i2u):
    s0, d0 = _prep_edges(ei_u2i)
    s1, d1 = _prep_edges(ei_i2u)
    zfeat = jnp.zeros((CHUNK, D), jnp.float32)
    ones = jnp.ones((CHUNK, D), jnp.float32)

    agg_item, agg_user, cnt_item, cnt_user = _sc_layer0(
        x_user, x_item, s0, d0, s1, d1, zfeat, ones)

    h_item = _tc_update(agg_item[:N], cnt_item[:N], x_item,
                        W_l_l0_u2i, W_r_l0_u2i, b_l0_u2i, relu=True)
    h_user = _tc_update(agg_user[:N], cnt_user[:N], x_user,
                        W_l_l0_i2u, W_r_l0_i2u, b_l0_i2u, relu=True)

    agg_item1, agg_user1 = _sc_layer1(h_user, h_item, s0, d0, s1, d1, zfeat)

    out_item = _tc_update(agg_item1[:N], cnt_item[:N], h_item,
                          W_l_l1_u2i, W_r_l1_u2i, b_l1_u2i, relu=False)
    out_user = _tc_update(agg_user1[:N], cnt_user[:N], h_user,
                          W_l_l1_i2u, W_r_l1_i2u, b_l1_i2u, relu=False)
    return (out_user, out_item)


# speedup vs baseline: 3.6703x; 1.0562x over previous
import functools

import jax
import jax.numpy as jnp
from jax import lax
from jax.experimental import pallas as pl
from jax.experimental.pallas import tpu as pltpu
from jax.experimental.pallas import tpu_sc as plsc

N = 10000
D = 128
E = 320000
NTILE = 16
CHUNK = 96
K = 216
KB = 24
NSTAGE = K // KB
EPAD = NTILE * K * CHUNK
ROWS_PT = 632
NPAD = NTILE * ROWS_PT

_SPANS = []
_r = 0
while _r < ROWS_PT:
    _SPANS.append((_r, min(CHUNK, ROWS_PT - _r)))
    _r += CHUNK


def _sc_body(with_counts, *refs):
    if with_counts:
        (tab0, tab1, s0, d0, s1, d1, zfeat, ones_hbm,
         out0, out1, cnt0, cnt1,
         acc, sidx, didx, rows, rows1, sem, sem1) = refs
    else:
        (tab0, tab1, s0, d0, s1, d1, zfeat,
         out0, out1,
         acc, sidx, didx, rows, rows1, sem, sem1) = refs
        cnt0 = cnt1 = None

    core = lax.axis_index("c")
    sid = lax.axis_index("s")
    rbase = sid * ROWS_PT

    def zero_acc_slice():
        pltpu.sync_copy(zfeat, rows)
        for (o, l) in _SPANS:
            pltpu.sync_copy(rows.at[pl.ds(0, l)], acc.at[pl.ds(rbase + o, l)])

    def copy_out(dst_hbm):
        for (o, l) in _SPANS:
            pltpu.sync_copy(acc.at[pl.ds(rbase + o, l)], rows.at[pl.ds(0, l)])
            pltpu.sync_copy(rows.at[pl.ds(0, l)], dst_hbm.at[pl.ds(rbase + o, l)])

    def run_dir(tab, srcr, dstr, out, cntout):
        if with_counts:
            zero_acc_slice()
            pltpu.sync_copy(ones_hbm, rows)
            plsc.subcore_barrier()

            def cstage(s, carry):
                pltpu.sync_copy(dstr.at[pl.ds(sid * K + s * KB, KB)], didx)

                def cbody(j, c2):
                    pltpu.sync_copy(rows, acc.at[didx.at[j]], add=True)
                    return c2

                return lax.fori_loop(0, KB, cbody, carry)

            lax.fori_loop(0, NSTAGE, cstage, 0)
            plsc.subcore_barrier()
            copy_out(cntout)

        zero_acc_slice()
        plsc.subcore_barrier()

        def stage(s, carry):
            base = sid * K + s * KB
            pltpu.sync_copy(srcr.at[pl.ds(base, KB)], sidx)
            pltpu.sync_copy(dstr.at[pl.ds(base, KB)], didx)
            pltpu.async_copy(tab.at[sidx.at[0]], rows, sem)

            def body(t, c2):
                j0 = 2 * t
                j1 = j0 + 1
                pltpu.async_copy(tab.at[sidx.at[j1]], rows1, sem1)
                pltpu.make_async_copy(tab.at[sidx.at[j0]], rows, sem).wait()
                pltpu.sync_copy(rows, acc.at[didx.at[j0]], add=True)

                @pl.when(j0 + 2 < KB)
                def _():
                    pltpu.async_copy(tab.at[sidx.at[j0 + 2]], rows, sem)

                pltpu.make_async_copy(tab.at[sidx.at[j1]], rows1, sem1).wait()
                pltpu.sync_copy(rows1, acc.at[didx.at[j1]], add=True)
                return c2

            return lax.fori_loop(0, KB // 2, body, carry)

        lax.fori_loop(0, NSTAGE, stage, 0)
        plsc.subcore_barrier()
        copy_out(out)

    @pl.when(core == 0)
    def _():
        run_dir(tab0, s0, d0, out0, cnt0)

    @pl.when(core == 1)
    def _():
        run_dir(tab1, s1, d1, out1, cnt1)


def _make_sc_call(with_counts):
    n_out = 4 if with_counts else 2
    out_type = [jax.ShapeDtypeStruct((NPAD, D), jnp.float32)] * n_out
    scratch = [
        pltpu.VMEM_SHARED((NPAD, D), jnp.float32),
        pltpu.VMEM((KB, CHUNK), jnp.int32),
        pltpu.VMEM((KB, CHUNK), jnp.int32),
        pltpu.VMEM((CHUNK, D), jnp.float32),
        pltpu.VMEM((CHUNK, D), jnp.float32),
        pltpu.SemaphoreType.DMA,
        pltpu.SemaphoreType.DMA,
    ]
    mesh = plsc.VectorSubcoreMesh(core_axis_name="c", subcore_axis_name="s")
    return pl.kernel(
        functools.partial(_sc_body, with_counts),
        out_type=out_type,
        mesh=mesh,
        scratch_types=scratch,
    )


_sc_layer0 = _make_sc_call(True)
_sc_layer1 = _make_sc_call(False)


def _tc_body(relu, agg_ref, cnt_ref, x_ref, wl_ref, wr_ref, b_ref, o_ref):
    c = jnp.maximum(cnt_ref[:, 0:1], 1.0)
    mean = agg_ref[...] / c
    acc = jnp.dot(mean, wl_ref[...], preferred_element_type=jnp.float32)
    acc = acc + jnp.dot(x_ref[...], wr_ref[...], preferred_element_type=jnp.float32)
    acc = acc + b_ref[...]
    if relu:
        acc = jnp.maximum(acc, 0.0)
    o_ref[...] = acc


_TC_BLK = 1000


def _tc_update(agg, cnt, x_dst, W_l, W_r, b, relu):
    grid = (N // _TC_BLK,)
    return pl.pallas_call(
        functools.partial(_tc_body, relu),
        grid=grid,
        in_specs=[
            pl.BlockSpec((_TC_BLK, D), lambda i: (i, 0)),
            pl.BlockSpec((_TC_BLK, D), lambda i: (i, 0)),
            pl.BlockSpec((_TC_BLK, D), lambda i: (i, 0)),
            pl.BlockSpec((D, D), lambda i: (0, 0)),
            pl.BlockSpec((D, D), lambda i: (0, 0)),
            pl.BlockSpec((1, D), lambda i: (0, 0)),
        ],
        out_specs=pl.BlockSpec((_TC_BLK, D), lambda i: (i, 0)),
        out_shape=jax.ShapeDtypeStruct((N, D), jnp.float32),
    )(agg, cnt, x_dst, W_l, W_r, b.reshape(1, D))


def _prep_edges(ei):
    src = ei[0].astype(jnp.int32)
    dst = ei[1].astype(jnp.int32)
    pad = EPAD - E
    src = jnp.concatenate([src, jnp.zeros((pad,), jnp.int32)])
    dst = jnp.concatenate([dst, jnp.full((pad,), N, jnp.int32)])
    return src.reshape(NTILE * K, CHUNK), dst.reshape(NTILE * K, CHUNK)


def kernel(x_user, x_item, ei_u2i, ei_i2u,
           W_l_l0_u2i, W_r_l0_u2i, b_l0_u2i,
           W_l_l0_i2u, W_r_l0_i2u, b_l0_i2u,
           W_l_l1_u2i, W_r_l1_u2i, b_l1_u2i,
           W_l_l1_i2u, W_r_l1_i2u, b_l1_i2u):
    s0, d0 = _prep_edges(ei_u2i)
    s1, d1 = _prep_edges(ei_i2u)
    zfeat = jnp.zeros((CHUNK, D), jnp.float32)
    ones = jnp.ones((CHUNK, D), jnp.float32)

    agg_item, agg_user, cnt_item, cnt_user = _sc_layer0(
        x_user, x_item, s0, d0, s1, d1, zfeat, ones)

    h_item = _tc_update(agg_item[:N], cnt_item[:N], x_item,
                        W_l_l0_u2i, W_r_l0_u2i, b_l0_u2i, relu=True)
    h_user = _tc_update(agg_user[:N], cnt_user[:N], x_user,
                        W_l_l0_i2u, W_r_l0_i2u, b_l0_i2u, relu=True)

    agg_item1, agg_user1 = _sc_layer1(h_user, h_item, s0, d0, s1, d1, zfeat)

    out_item = _tc_update(agg_item1[:N], cnt_item[:N], h_item,
                          W_l_l1_u2i, W_r_l1_u2i, b_l1_u2i, relu=False)
    out_user = _tc_update(agg_user1[:N], cnt_user[:N], h_user,
                          W_l_l1_i2u, W_r_l1_i2u, b_l1_i2u, relu=False)
    return (out_user, out_item)


# speedup vs baseline: 6.1734x; 1.6820x over previous
import functools

import jax
import jax.numpy as jnp
from jax import lax
from jax.experimental import pallas as pl
from jax.experimental.pallas import tpu as pltpu
from jax.experimental.pallas import tpu_sc as plsc

N = 10000
D = 128
E = 320000
NTILE = 16
CHUNK = 120
K = 168
KB = 8
NBUF = 2
NSTAGE = K // KB
EPAD = NTILE * K * CHUNK
ROWS_PT = 632
NPAD = NTILE * ROWS_PT

_SPANS = []
_r = 0
while _r < ROWS_PT:
    _SPANS.append((_r, min(CHUNK, ROWS_PT - _r)))
    _r += CHUNK


def _sc_body(with_counts, *refs):
    if with_counts:
        (tab0, tab1, s0, d0, s1, d1, zfeat, ones_hbm,
         out0, out1, cnt0, cnt1,
         acc, sidx, didx, *bufsems) = refs
    else:
        (tab0, tab1, s0, d0, s1, d1, zfeat,
         out0, out1,
         acc, sidx, didx, *bufsems) = refs
        cnt0 = cnt1 = None
    bufs = bufsems[:NBUF]
    sems = bufsems[NBUF:]
    rows = bufs[0]

    core = lax.axis_index("c")
    sid = lax.axis_index("s")
    rbase = sid * ROWS_PT

    def zero_acc_slice():
        pltpu.sync_copy(zfeat, rows)
        for (o, l) in _SPANS:
            pltpu.sync_copy(rows.at[pl.ds(0, l)], acc.at[pl.ds(rbase + o, l)])

    def copy_out(dst_hbm):
        for (o, l) in _SPANS:
            pltpu.sync_copy(acc.at[pl.ds(rbase + o, l)], rows.at[pl.ds(0, l)])
            pltpu.sync_copy(rows.at[pl.ds(0, l)], dst_hbm.at[pl.ds(rbase + o, l)])

    def run_dir(tab, srcr, dstr, out, cntout):
        if with_counts:
            zero_acc_slice()
            pltpu.sync_copy(ones_hbm, rows)
            plsc.subcore_barrier()

            def cstage(s, carry):
                pltpu.sync_copy(dstr.at[pl.ds(sid * K + s * KB, KB)], didx)

                def cbody(j, c2):
                    pltpu.sync_copy(rows, acc.at[didx.at[j]], add=True)
                    return c2

                return lax.fori_loop(0, KB, cbody, carry)

            lax.fori_loop(0, NSTAGE, cstage, 0)
            plsc.subcore_barrier()
            copy_out(cntout)

        zero_acc_slice()
        plsc.subcore_barrier()

        def pump(j, b):
            @pl.when(j + NBUF < KB)
            def _():
                pltpu.async_copy(tab.at[sidx.at[j + NBUF]], bufs[b], sems[b])

        def stage(s, carry):
            base = sid * K + s * KB
            pltpu.sync_copy(srcr.at[pl.ds(base, KB)], sidx)
            pltpu.sync_copy(dstr.at[pl.ds(base, KB)], didx)
            for b in range(NBUF):
                pltpu.async_copy(tab.at[sidx.at[b]], bufs[b], sems[b])

            def body(t, c2):
                for b in range(NBUF):
                    j = NBUF * t + b
                    pltpu.make_async_copy(tab.at[sidx.at[j]],
                                          bufs[b], sems[b]).wait()
                    pltpu.sync_copy(bufs[b], acc.at[didx.at[j]], add=True)
                    pump(j, b)
                return c2

            return lax.fori_loop(0, KB // NBUF, body, carry)

        lax.fori_loop(0, NSTAGE, stage, 0)
        plsc.subcore_barrier()
        copy_out(out)

    @pl.when(core == 0)
    def _():
        run_dir(tab0, s0, d0, out0, cnt0)

    @pl.when(core == 1)
    def _():
        run_dir(tab1, s1, d1, out1, cnt1)


def _make_sc_call(with_counts):
    n_out = 4 if with_counts else 2
    out_type = [jax.ShapeDtypeStruct((NPAD, D), jnp.float32)] * n_out
    scratch = [
        pltpu.VMEM_SHARED((NPAD, D), jnp.float32),
        pltpu.VMEM((KB, CHUNK), jnp.int32),
        pltpu.VMEM((KB, CHUNK), jnp.int32),
    ]
    scratch += [pltpu.VMEM((CHUNK, D), jnp.float32)] * NBUF
    scratch += [pltpu.SemaphoreType.DMA] * NBUF
    mesh = plsc.VectorSubcoreMesh(core_axis_name="c", subcore_axis_name="s")
    return pl.kernel(
        functools.partial(_sc_body, with_counts),
        out_type=out_type,
        mesh=mesh,
        scratch_types=scratch,
    )


_sc_layer0 = _make_sc_call(True)
_sc_layer1 = _make_sc_call(False)


def _tc_body(relu, agg_ref, cnt_ref, x_ref, wl_ref, wr_ref, b_ref, o_ref):
    c = jnp.maximum(cnt_ref[:, 0:1], 1.0)
    mean = agg_ref[...] / c
    acc = jnp.dot(mean, wl_ref[...], preferred_element_type=jnp.float32)
    acc = acc + jnp.dot(x_ref[...], wr_ref[...], preferred_element_type=jnp.float32)
    acc = acc + b_ref[...]
    if relu:
        acc = jnp.maximum(acc, 0.0)
    o_ref[...] = acc


_TC_BLK = 1000


def _tc_update(agg, cnt, x_dst, W_l, W_r, b, relu):
    grid = (N // _TC_BLK,)
    return pl.pallas_call(
        functools.partial(_tc_body, relu),
        grid=grid,
        in_specs=[
            pl.BlockSpec((_TC_BLK, D), lambda i: (i, 0)),
            pl.BlockSpec((_TC_BLK, D), lambda i: (i, 0)),
            pl.BlockSpec((_TC_BLK, D), lambda i: (i, 0)),
            pl.BlockSpec((D, D), lambda i: (0, 0)),
            pl.BlockSpec((D, D), lambda i: (0, 0)),
            pl.BlockSpec((1, D), lambda i: (0, 0)),
        ],
        out_specs=pl.BlockSpec((_TC_BLK, D), lambda i: (i, 0)),
        out_shape=jax.ShapeDtypeStruct((N, D), jnp.float32),
    )(agg, cnt, x_dst, W_l, W_r, b.reshape(1, D))


def _prep_edges(ei):
    src = ei[0].astype(jnp.int32)
    dst = ei[1].astype(jnp.int32)
    pad = EPAD - E
    src = jnp.concatenate([src, jnp.zeros((pad,), jnp.int32)])
    dst = jnp.concatenate([dst, jnp.full((pad,), N, jnp.int32)])
    return src.reshape(NTILE * K, CHUNK), dst.reshape(NTILE * K, CHUNK)


def kernel(x_user, x_item, ei_u2i, ei_i2u,
           W_l_l0_u2i, W_r_l0_u2i, b_l0_u2i,
           W_l_l0_i2u, W_r_l0_i2u, b_l0_i2u,
           W_l_l1_u2i, W_r_l1_u2i, b_l1_u2i,
           W_l_l1_i2u, W_r_l1_i2u, b_l1_i2u):
    s0, d0 = _prep_edges(ei_u2i)
    s1, d1 = _prep_edges(ei_i2u)
    zfeat = jnp.zeros((CHUNK, D), jnp.float32)
    ones = jnp.ones((CHUNK, D), jnp.float32)

    agg_item, agg_user, cnt_item, cnt_user = _sc_layer0(
        x_user, x_item, s0, d0, s1, d1, zfeat, ones)

    h_item = _tc_update(agg_item[:N], cnt_item[:N], x_item,
                        W_l_l0_u2i, W_r_l0_u2i, b_l0_u2i, relu=True)
    h_user = _tc_update(agg_user[:N], cnt_user[:N], x_user,
                        W_l_l0_i2u, W_r_l0_i2u, b_l0_i2u, relu=True)

    agg_item1, agg_user1 = _sc_layer1(h_user, h_item, s0, d0, s1, d1, zfeat)

    out_item = _tc_update(agg_item1[:N], cnt_item[:N], h_item,
                          W_l_l1_u2i, W_r_l1_u2i, b_l1_u2i, relu=False)
    out_user = _tc_update(agg_user1[:N], cnt_user[:N], h_user,
                          W_l_l1_i2u, W_r_l1_i2u, b_l1_i2u, relu=False)
    return (out_user, out_item)
